# baseline (speedup 1.0000x reference)
import jax
import jax.numpy as jnp
from jax.experimental import pallas as pl
from jax.experimental.pallas import tpu as pltpu

H = 128


def _fused_lstm_kernel(idx_hbm, emb_hbm, h_ref, c_ref, wih_ref, whh_ref,
                       bih_ref, bhh_ref, hn_ref, cn_ref,
                       idx_smem, x_vmem, sem_i, sem_x):
    cp_i = pltpu.make_async_copy(idx_hbm, idx_smem, sem_i)
    cp_i.start()
    cp_i.wait()
    row = idx_smem[0]
    cp_x = pltpu.make_async_copy(emb_hbm.at[pl.ds(row, 1), :], x_vmem, sem_x)
    cp_x.start()
    cp_x.wait()
    x = x_vmem[...]
    h = h_ref[0]
    c = c_ref[0]
    dn = (((1,), (1,)), ((), ()))
    gates = jax.lax.dot_general(x, wih_ref[...], dn,
                                preferred_element_type=jnp.float32)
    gates = gates + jax.lax.dot_general(h, whh_ref[...], dn,
                                        preferred_element_type=jnp.float32)
    gates = gates + (bih_ref[...] + bhh_ref[...])[None, :]
    i = jax.nn.sigmoid(gates[:, 0 * H:1 * H])
    f = jax.nn.sigmoid(gates[:, 1 * H:2 * H])
    g = jnp.tanh(gates[:, 2 * H:3 * H])
    o = jax.nn.sigmoid(gates[:, 3 * H:4 * H])
    cn = f * c + i * g
    hn_ref[0] = o * jnp.tanh(cn)
    cn_ref[0] = cn


def kernel(word_num, hidden, cell, emb, W_ih, W_hh, b_ih, b_hh):
    idx = jnp.asarray(word_num, jnp.int32).reshape(1)
    hn, cn = pl.pallas_call(
        _fused_lstm_kernel,
        in_specs=[
            pl.BlockSpec(memory_space=pl.ANY),
            pl.BlockSpec(memory_space=pl.ANY),
            pl.BlockSpec((1, 1, H), lambda: (0, 0, 0)),
            pl.BlockSpec((1, 1, H), lambda: (0, 0, 0)),
            pl.BlockSpec((4 * H, H), lambda: (0, 0)),
            pl.BlockSpec((4 * H, H), lambda: (0, 0)),
            pl.BlockSpec((4 * H,), lambda: (0,)),
            pl.BlockSpec((4 * H,), lambda: (0,)),
        ],
        out_specs=[
            pl.BlockSpec((1, 1, H), lambda: (0, 0, 0)),
            pl.BlockSpec((1, 1, H), lambda: (0, 0, 0)),
        ],
        out_shape=[
            jax.ShapeDtypeStruct((1, 1, H), jnp.float32),
            jax.ShapeDtypeStruct((1, 1, H), jnp.float32),
        ],
        scratch_shapes=[
            pltpu.SMEM((1,), jnp.int32),
            pltpu.VMEM((1, H), jnp.float32),
            pltpu.SemaphoreType.DMA,
            pltpu.SemaphoreType.DMA,
        ],
    )(idx, emb, hidden, cell, W_ih, W_hh, b_ih, b_hh)
    return (hn, hn, cn)
